# baseline (speedup 1.0000x reference)
import functools

import jax
import jax.numpy as jnp
from jax import lax
from jax.experimental import pallas as pl
from jax.experimental.pallas import tpu as pltpu
from jax.experimental.pallas import tpu_sc as plsc

B = 16384
S = 50
DIM = 64
NUM_TOK = B * S
NC = 2
NS = 16
NW = NC * NS
BPW = B // NW
PER_W = BPW * S
CB = 256
NCH = S * (BPW // CB)
L = 16

_mesh = plsc.VectorSubcoreMesh(core_axis_name="c", subcore_axis_name="s")


@functools.partial(
    pl.kernel,
    mesh=_mesh,
    out_type=jax.ShapeDtypeStruct((S, DIM, B), jnp.float32),
    scratch_types=[
        pltpu.VMEM((PER_W,), jnp.int32),
        pltpu.VMEM((CB, DIM), jnp.float32),
        pltpu.VMEM((CB, DIM), jnp.float32),
        pltpu.VMEM((DIM, CB), jnp.float32),
        pltpu.VMEM((DIM, CB), jnp.float32),
        pltpu.SemaphoreType.DMA,
        pltpu.SemaphoreType.DMA,
        pltpu.SemaphoreType.DMA,
        pltpu.SemaphoreType.DMA,
    ],
    compiler_params=pltpu.CompilerParams(
        use_tc_tiling_on_sc=False, needs_layout_passes=False),
)
def _gather_t(idx_hbm, table_hbm, out_hbm, idx_v, rows0, rows1, tr0, tr1,
              gsem0, gsem1, ssem0, ssem1):
    wid = lax.axis_index("s") * NC + lax.axis_index("c")
    ibase = wid * PER_W
    bbase = wid * BPW
    rows = (rows0, rows1)
    trows = (tr0, tr1)
    gsem = (gsem0, gsem1)
    ssem = (ssem0, ssem1)

    pltpu.sync_copy(idx_hbm.at[pl.ds(ibase, PER_W)], idx_v)

    def fire_gather(ch, b):
        pltpu.async_copy(
            table_hbm.at[idx_v.at[pl.ds(ch * CB, CB)]],
            rows[b], gsem[b])

    def drain_gather(b):
        pltpu.make_async_copy(
            table_hbm.at[pl.ds(0, CB)], rows[b], gsem[b]).wait()

    def transpose(b):
        lanes = lax.iota(jnp.int32, L)

        def tb_body(tb, carry):
            row_idx = tb * L + lanes
            for col in range(DIM):
                vec = plsc.load_gather(
                    rows[b], [row_idx, jnp.full((L,), col, jnp.int32)])
                trows[b][col, pl.ds(tb * L, L)] = vec
            return carry

        lax.fori_loop(0, CB // L, tb_body, 0)

    def fire_store(ch, b):
        s = ch // 2
        h = ch % 2
        pltpu.async_copy(
            trows[b],
            out_hbm.at[s, pl.ds(0, DIM), pl.ds(bbase + h * CB, CB)],
            ssem[b])

    def drain_store(b):
        pltpu.make_async_copy(
            table_hbm.at[pl.ds(0, DIM)], trows[b], ssem[b]).wait()

    def body(g, carry):
        for b in (0, 1):
            ch = 2 * g + b
            fire_gather(ch, b)
        for b in (0, 1):
            ch = 2 * g + b
            drain_gather(b)
            @pl.when(g >= 1)
            def _():
                drain_store(b)
            transpose(b)
            fire_store(ch, b)
        return carry

    lax.fori_loop(0, NCH // 2, body, 0)
    drain_store(0)
    drain_store(1)


def kernel(tokens_ids, embedding_tensor):
    idxp = (tokens_ids.T.reshape(S, NW, BPW)
            .transpose(1, 0, 2).reshape(-1).astype(jnp.int32))
    out_t = _gather_t(idxp, embedding_tensor)
    return out_t.transpose(2, 0, 1)


# speedup vs baseline: 1.3506x; 1.3506x over previous
import functools

import jax
import jax.numpy as jnp
from jax import lax
from jax.experimental import pallas as pl
from jax.experimental.pallas import tpu as pltpu
from jax.experimental.pallas import tpu_sc as plsc

B = 16384
S = 50
DIM = 64
NUM_TOK = B * S
NC = 2
NS = 16
NW = NC * NS
PER_W = NUM_TOK // NW
BPW = B // NW
C = 400
CB = C // S
NCH = PER_W // C

_mesh = plsc.VectorSubcoreMesh(core_axis_name="c", subcore_axis_name="s")


@functools.partial(
    pl.kernel,
    mesh=_mesh,
    out_type=jax.ShapeDtypeStruct((B, S, DIM), jnp.float32),
    scratch_types=[
        pltpu.VMEM((PER_W,), jnp.int32),
        pltpu.VMEM((C, DIM), jnp.float32),
        pltpu.VMEM((C, DIM), jnp.float32),
        pltpu.SemaphoreType.DMA,
        pltpu.SemaphoreType.DMA,
        pltpu.SemaphoreType.DMA,
        pltpu.SemaphoreType.DMA,
    ],
    compiler_params=pltpu.CompilerParams(use_tc_tiling_on_sc=False),
)
def _gather(idx_hbm, table_hbm, out_hbm, idx_v, rows0, rows1,
            gsem0, gsem1, ssem0, ssem1):
    wid = lax.axis_index("s") * NC + lax.axis_index("c")
    ibase = wid * PER_W
    bbase = wid * BPW
    rows = (rows0, rows1)
    gsem = (gsem0, gsem1)
    ssem = (ssem0, ssem1)

    pltpu.sync_copy(idx_hbm.at[pl.ds(ibase, PER_W)], idx_v)

    def fire_gather(c, b):
        pltpu.async_copy(
            table_hbm.at[idx_v.at[pl.ds(c * C, C)]],
            rows[b], gsem[b])

    def drain_gather(b):
        pltpu.make_async_copy(
            table_hbm.at[pl.ds(0, C)], rows[b], gsem[b]).wait()

    def fire_stores(c, b):
        for k in range(CB):
            pltpu.async_copy(
                rows[b].at[pl.ds(k * S, S)],
                out_hbm.at[bbase + c * CB + k],
                ssem[b])

    def drain_stores(b):
        for k in range(CB):
            pltpu.make_async_copy(
                table_hbm.at[pl.ds(0, S)],
                rows[b].at[pl.ds(k * S, S)],
                ssem[b]).wait()

    def body(g, carry):
        for b in (0, 1):
            c = 2 * g + b
            @pl.when(g >= 1)
            def _():
                drain_stores(b)
            fire_gather(c, b)
        for b in (0, 1):
            c = 2 * g + b
            drain_gather(b)
            fire_stores(c, b)
        return carry

    lax.fori_loop(0, NCH // 2, body, 0)
    drain_stores(0)
    drain_stores(1)


def kernel(tokens_ids, embedding_tensor):
    flat = tokens_ids.reshape(-1).astype(jnp.int32)
    one = lax.optimization_barrier(jnp.float32(1.0))
    t64 = embedding_tensor * one
    return _gather(flat, t64)


# speedup vs baseline: 1.6895x; 1.2510x over previous
import functools

import jax
import jax.numpy as jnp
from jax import lax
from jax.experimental import pallas as pl
from jax.experimental.pallas import tpu as pltpu
from jax.experimental.pallas import tpu_sc as plsc

B = 16384
S = 50
DIM = 64
NUM_TOK = B * S
NC = 2
NS = 16
NW = NC * NS
PER_W = NUM_TOK // NW
BPW = B // NW
C = 400
CB = C // S
NCH = PER_W // C

_mesh = plsc.VectorSubcoreMesh(core_axis_name="c", subcore_axis_name="s")


@functools.partial(
    pl.kernel,
    mesh=_mesh,
    out_type=jax.ShapeDtypeStruct((B, S, DIM), jnp.float32),
    scratch_types=[
        pltpu.VMEM((PER_W,), jnp.int32),
        pltpu.VMEM((C, DIM), jnp.float32),
        pltpu.VMEM((C, DIM), jnp.float32),
        pltpu.SemaphoreType.DMA,
        pltpu.SemaphoreType.DMA,
        pltpu.SemaphoreType.DMA,
        pltpu.SemaphoreType.DMA,
    ],
    compiler_params=pltpu.CompilerParams(use_tc_tiling_on_sc=False),
)
def _gather(idx_hbm, table_hbm, out_hbm, idx_v, rows0, rows1,
            gsem0, gsem1, ssem0, ssem1):
    wid = lax.axis_index("s") * NC + lax.axis_index("c")
    ibase = wid * PER_W
    bbase = wid * BPW
    rows = (rows0, rows1)
    gsem = (gsem0, gsem1)
    ssem = (ssem0, ssem1)

    pltpu.sync_copy(idx_hbm.at[pl.ds(ibase, PER_W)], idx_v)

    def fire_gather(c, b):
        pltpu.async_copy(
            table_hbm.at[idx_v.at[pl.ds(c * C, C)]],
            rows[b], gsem[b])

    def drain_gather(b):
        pltpu.make_async_copy(
            table_hbm.at[pl.ds(0, C)], rows[b], gsem[b]).wait()

    def fire_stores(c, b):
        for k in range(CB):
            pltpu.async_copy(
                rows[b].at[pl.ds(k * S, S)],
                out_hbm.at[bbase + c * CB + k],
                ssem[b])

    def drain_stores(b):
        for k in range(CB):
            pltpu.make_async_copy(
                table_hbm.at[pl.ds(0, S)],
                rows[b].at[pl.ds(k * S, S)],
                ssem[b]).wait()

    def body(g, carry):
        for b in (0, 1):
            c = 2 * g + b
            @pl.when(g >= 1)
            def _():
                drain_stores(b)
            fire_gather(c, b)
        for b in (0, 1):
            c = 2 * g + b
            drain_gather(b)
            fire_stores(c, b)
        return carry

    lax.fori_loop(0, NCH // 2, body, 0)
    drain_stores(0)
    drain_stores(1)


def kernel(tokens_ids, embedding_tensor):
    flat = tokens_ids.reshape(-1).astype(jnp.int32)
    return _gather(flat, embedding_tensor)
